# Initial kernel scaffold; baseline (speedup 1.0000x reference)
import jax
import jax.numpy as jnp
from jax.experimental import pallas as pl


def kernel(inputs, table):
    raise NotImplementedError("write your pallas kernel here")


# speedup vs baseline: 2.6531x; 2.6531x over previous
import functools

import jax
import jax.numpy as jnp
from jax import lax
from jax.experimental import pallas as pl
from jax.experimental.pallas import tpu as pltpu
from jax.experimental.pallas import tpu_sc as plsc

EMBED_DIM = 32
CHUNK = 3200


@functools.cache
def _make_gather(B: int, V: int, D: int):
    info = plsc.get_sparse_core_info()
    nc, ns = info.num_cores, info.num_subcores
    nw = nc * ns
    b_per_w = B // nw
    assert b_per_w * nw == B and b_per_w % CHUNK == 0
    n_chunks = b_per_w // CHUNK
    mesh = plsc.VectorSubcoreMesh(core_axis_name="c", subcore_axis_name="s")

    @functools.partial(
        pl.kernel,
        mesh=mesh,
        out_type=jax.ShapeDtypeStruct((B, D), jnp.float32),
        compiler_params=pltpu.CompilerParams(use_tc_tiling_on_sc=False),
        scratch_types=[
            pltpu.VMEM((CHUNK,), jnp.int32),
            pltpu.VMEM((CHUNK, D), jnp.float32),
            pltpu.SemaphoreType.DMA,
        ],
    )
    def gather_kernel(table_hbm, idx_hbm, out_hbm, idx_v, rows_v, sem):
        wid = lax.axis_index("s") * nc + lax.axis_index("c")
        base = wid * b_per_w

        def body(i, carry):
            off = base + i * CHUNK
            pltpu.sync_copy(idx_hbm.at[pl.ds(off, CHUNK)], idx_v)
            pltpu.async_copy(table_hbm.at[idx_v], rows_v, sem).wait()
            pltpu.sync_copy(rows_v, out_hbm.at[pl.ds(off, CHUNK)])
            return carry

        lax.fori_loop(0, n_chunks, body, 0)

    return gather_kernel


def kernel(inputs, table):
    batch, hist = inputs.shape
    rows, dim = table.shape
    flat_idx = inputs.reshape(batch * hist)
    out = _make_gather(batch * hist, rows, dim)(table, flat_idx)
    return out.reshape(batch, hist, dim)


# speedup vs baseline: 3.1688x; 1.1944x over previous
import functools

import jax
import jax.numpy as jnp
from jax import lax
from jax.experimental import pallas as pl
from jax.experimental.pallas import tpu as pltpu
from jax.experimental.pallas import tpu_sc as plsc

CHUNK = 640
NBUF = 4


@functools.cache
def _make_gather(B: int, V: int, D: int):
    info = plsc.get_sparse_core_info()
    nc, ns = info.num_cores, info.num_subcores
    nw = nc * ns
    b_per_w = B // nw
    assert b_per_w * nw == B and b_per_w % (CHUNK * NBUF) == 0
    n_blocks = b_per_w // (CHUNK * NBUF)
    mesh = plsc.VectorSubcoreMesh(core_axis_name="c", subcore_axis_name="s")

    @functools.partial(
        pl.kernel,
        mesh=mesh,
        out_type=jax.ShapeDtypeStruct((B, D), jnp.float32),
        compiler_params=pltpu.CompilerParams(use_tc_tiling_on_sc=False),
        scratch_types=[
            pltpu.VMEM_SHARED((V, D), jnp.float32),
            pltpu.VMEM((b_per_w,), jnp.int32),
            pltpu.VMEM((NBUF, CHUNK, D), jnp.float32),
            pltpu.SemaphoreType.DMA,
            pltpu.SemaphoreType.DMA((NBUF,)),
        ],
    )
    def gather_kernel(table_hbm, idx_hbm, out_hbm, table_sh, idx_v, rows_v,
                      gsem, ssem):
        cid = lax.axis_index("c")
        sid = lax.axis_index("s")
        wid = sid * nc + cid
        base = wid * b_per_w

        @pl.when(sid == 0)
        def _():
            pltpu.sync_copy(table_hbm, table_sh)

        pltpu.sync_copy(idx_hbm.at[pl.ds(base, b_per_w)], idx_v)
        plsc.subcore_barrier()

        def block(j, carry):
            for b in range(NBUF):
                i = j * NBUF + b

                @pl.when(j > 0)
                def _():
                    pltpu.make_async_copy(
                        rows_v.at[b], out_hbm.at[pl.ds(base, CHUNK)],
                        ssem.at[b]).wait()

                idx_c = idx_v.at[pl.ds(i * CHUNK, CHUNK)]
                pltpu.async_copy(table_sh.at[idx_c], rows_v.at[b], gsem).wait()
                pltpu.async_copy(
                    rows_v.at[b], out_hbm.at[pl.ds(base + i * CHUNK, CHUNK)],
                    ssem.at[b])
            return carry

        lax.fori_loop(0, n_blocks, block, 0)
        for b in range(NBUF):
            pltpu.make_async_copy(
                rows_v.at[b], out_hbm.at[pl.ds(base, CHUNK)], ssem.at[b]).wait()

    return gather_kernel


def kernel(inputs, table):
    batch, hist = inputs.shape
    rows, dim = table.shape
    flat_idx = inputs.reshape(batch * hist)
    out = _make_gather(batch * hist, rows, dim)(table, flat_idx)
    return out.reshape(batch, hist, dim)


# speedup vs baseline: 4.8584x; 1.5332x over previous
import functools

import jax
import jax.numpy as jnp
from jax import lax
from jax.experimental import pallas as pl
from jax.experimental.pallas import tpu as pltpu
from jax.experimental.pallas import tpu_sc as plsc

L = 16


@functools.cache
def _make_gather(B: int, H: int, V: int, D: int):
    info = plsc.get_sparse_core_info()
    nc, ns = info.num_cores, info.num_subcores
    nw = nc * ns
    nb = B // nw
    nbt = nb // 128
    dt, di = D // 8, 8
    assert nb * nw == B and nbt * 128 == nb
    mesh = plsc.VectorSubcoreMesh(core_axis_name="c", subcore_axis_name="s")

    @functools.partial(
        pl.kernel,
        mesh=mesh,
        out_type=jax.ShapeDtypeStruct((H, dt, B // 128, di, 128), jnp.float32),
        compiler_params=pltpu.CompilerParams(
            use_tc_tiling_on_sc=False, needs_layout_passes=False),
        scratch_types=[
            pltpu.VMEM_SHARED((V, D), jnp.float32),
            pltpu.VMEM((nb * H,), jnp.int32),
            pltpu.VMEM((2, nb), jnp.int32),
            pltpu.VMEM((2, nb, D), jnp.float32),
            pltpu.VMEM((2, dt, nbt, di, 128), jnp.float32),
            pltpu.SemaphoreType.DMA((2,)),
            pltpu.SemaphoreType.DMA((2,)),
        ],
    )
    def gather_kernel(table_hbm, idx_hbm, out_hbm, table_sh, idx_v, idxh_v,
                      rows_v, stg_v, gsem, ssem):
        cid = lax.axis_index("c")
        sid = lax.axis_index("s")
        wid = sid * nc + cid
        base = wid * nb * H

        @pl.when(sid == 0)
        def _():
            pltpu.sync_copy(table_hbm, table_sh)

        pltpu.sync_copy(idx_hbm.at[pl.ds(base, nb * H)], idx_v)
        plsc.subcore_barrier()

        iota = lax.iota(jnp.int32, L)
        iota_h = iota * H

        def build_idxh(p, h):
            for g in range(nb // L):
                vec = iota_h + (g * L * H + h)
                idxh_v[p, pl.ds(g * L, L)] = plsc.load_gather(idx_v, [vec])

        def start_gather(p):
            pltpu.async_copy(table_sh.at[idxh_v.at[p]], rows_v.at[p],
                             gsem.at[p])

        def wait_gather(p):
            pltpu.make_async_copy(table_sh.at[idxh_v.at[p]], rows_v.at[p],
                                  gsem.at[p]).wait()

        def transpose(p):
            for btl in range(nbt):
                for g in range(128 // L):
                    rowvec = iota + (btl * 128 + g * L)
                    for d in range(D):
                        vals = plsc.load_gather(
                            rows_v.at[p],
                            [rowvec, jnp.full((L,), d, jnp.int32)])
                        stg_v[p, d // di, btl, d % di, pl.ds(g * L, L)] = vals

        def store_dst(h):
            return out_hbm.at[h, :, pl.ds(wid * nbt, nbt)]

        def start_store(p, h):
            pltpu.async_copy(stg_v.at[p], store_dst(h), ssem.at[p])

        def wait_store(p, h):
            pltpu.make_async_copy(stg_v.at[p], store_dst(h),
                                  ssem.at[p]).wait()

        build_idxh(0, 0)
        start_gather(0)

        def block(j, carry):
            for p in range(2):
                h = 2 * j + p
                wait_gather(p)
                if p == 0:
                    build_idxh(1, h + 1)
                    start_gather(1)
                else:
                    @pl.when(j < (H // 2) - 1)
                    def _():
                        build_idxh(0, h + 1)
                        start_gather(0)

                @pl.when(j > 0)
                def _():
                    wait_store(p, h - 2)

                transpose(p)
                start_store(p, h)
            return carry

        lax.fori_loop(0, H // 2, block, 0)
        wait_store(0, H - 2)
        wait_store(1, H - 1)

    return gather_kernel


def kernel(inputs, table):
    batch, hist = inputs.shape
    rows, dim = table.shape
    flat_idx = inputs.reshape(batch * hist)
    out5 = _make_gather(batch, hist, rows, dim)(table, flat_idx)
    out = jnp.transpose(out5, (2, 4, 0, 1, 3))
    return out.reshape(batch, hist, dim)


# speedup vs baseline: 8.7043x; 1.7916x over previous
import functools

import jax
import jax.numpy as jnp
from jax import lax
from jax.experimental import pallas as pl
from jax.experimental.pallas import tpu as pltpu
from jax.experimental.pallas import tpu_sc as plsc

L = 16


W = 40


@functools.cache
def _make_gather(B: int, H: int, V: int, D: int):
    info = plsc.get_sparse_core_info()
    nc, ns = info.num_cores, info.num_subcores
    nw = nc * ns
    nb = B // nw
    nbt = nb // 128
    dt, di = D // 8, 8
    assert nb * nw == B and nbt * 128 == nb
    mesh = plsc.VectorSubcoreMesh(core_axis_name="c", subcore_axis_name="s")

    @functools.partial(
        pl.kernel,
        mesh=mesh,
        out_type=jax.ShapeDtypeStruct((H, dt, B // 128, di, 128), jnp.float32),
        compiler_params=pltpu.CompilerParams(
            use_tc_tiling_on_sc=False, needs_layout_passes=False),
        scratch_types=[
            pltpu.VMEM_SHARED((V, W), jnp.float32),
            pltpu.VMEM((nb * H,), jnp.int32),
            pltpu.VMEM((2, nb), jnp.int32),
            pltpu.VMEM((2, nb, W), jnp.float32),
            pltpu.VMEM((2, dt, nbt, di, 128), jnp.float32),
            pltpu.SemaphoreType.DMA((2,)),
            pltpu.SemaphoreType.DMA((2,)),
        ],
    )
    def gather_kernel(table_hbm, idx_hbm, out_hbm, table_sh, idx_v, idxh_v,
                      rows_v, stg_v, gsem, ssem):
        cid = lax.axis_index("c")
        sid = lax.axis_index("s")
        wid = sid * nc + cid

        @pl.when(sid == 0)
        def _():
            pltpu.sync_copy(table_hbm, table_sh)

        pltpu.sync_copy(idx_hbm.at[pl.ds(wid * nb * H, nb * H)], idx_v)
        plsc.subcore_barrier()

        iota = lax.iota(jnp.int32, L)
        iota_h = iota * H

        def build_idxh(p, h):
            for g in range(nb // L):
                vec = iota_h + (g * L * H + h)
                idxh_v[p, pl.ds(g * L, L)] = plsc.load_gather(idx_v, [vec])

        def start_gather(p):
            pltpu.async_copy(table_sh.at[idxh_v.at[p]], rows_v.at[p],
                             gsem.at[p])

        def wait_gather(p):
            pltpu.make_async_copy(table_sh.at[idxh_v.at[p]], rows_v.at[p],
                                  gsem.at[p]).wait()

        def transpose(p):
            for btl in range(nbt):
                for g in range(128 // L):
                    rowvec = iota + (btl * 128 + g * L)
                    for d in range(D):
                        vals = plsc.load_gather(
                            rows_v.at[p],
                            [rowvec, jnp.full((L,), d, jnp.int32)])
                        stg_v[p, d // di, btl, d % di, pl.ds(g * L, L)] = vals

        def store_dst(h):
            return out_hbm.at[h, :, pl.ds(wid * nbt, nbt)]

        def start_store(p, h):
            pltpu.async_copy(stg_v.at[p], store_dst(h), ssem.at[p])

        def wait_store(p, h):
            pltpu.make_async_copy(stg_v.at[p], store_dst(h),
                                  ssem.at[p]).wait()

        build_idxh(0, 0)
        start_gather(0)

        def block(j, carry):
            for p in range(2):
                h = 2 * j + p
                wait_gather(p)
                if p == 0:
                    build_idxh(1, h + 1)
                    start_gather(1)
                else:
                    @pl.when(j < (H // 2) - 1)
                    def _():
                        build_idxh(0, h + 1)
                        start_gather(0)

                @pl.when(j > 0)
                def _():
                    wait_store(p, h - 2)

                transpose(p)
                start_store(p, h)
            return carry

        lax.fori_loop(0, H // 2, block, 0)
        wait_store(0, H - 2)
        wait_store(1, H - 1)

    return gather_kernel


def kernel(inputs, table):
    batch, hist = inputs.shape
    rows, dim = table.shape
    flat_idx = inputs.reshape(batch * hist)
    table_p = jnp.pad(table, ((0, 0), (0, W - dim)))
    out5 = _make_gather(batch, hist, rows, dim)(table_p, flat_idx)
    out = jnp.transpose(out5, (2, 4, 0, 1, 3))
    return out.reshape(batch, hist, dim)


# speedup vs baseline: 9.1812x; 1.0548x over previous
import functools

import jax
import jax.numpy as jnp
from jax import lax
from jax.experimental import pallas as pl
from jax.experimental.pallas import tpu as pltpu
from jax.experimental.pallas import tpu_sc as plsc

L = 16


@functools.cache
def _make_gather(B: int, H: int, V: int, D: int):
    info = plsc.get_sparse_core_info()
    nc, ns = info.num_cores, info.num_subcores
    nw = nc * ns
    nb = B // nw
    nbt = nb // 128
    dt, di = D // 8, 8
    assert nb * nw == B and nbt * 128 == nb
    mesh = plsc.VectorSubcoreMesh(core_axis_name="c", subcore_axis_name="s")

    @functools.partial(
        pl.kernel,
        mesh=mesh,
        out_type=jax.ShapeDtypeStruct((H, dt, B // 128, di, 128), jnp.float32),
        compiler_params=pltpu.CompilerParams(
            use_tc_tiling_on_sc=False, needs_layout_passes=False),
        scratch_types=[
            pltpu.VMEM((D, V), jnp.float32),
            pltpu.VMEM((nb * H,), jnp.int32),
            pltpu.VMEM((2, dt, nbt, di, 128), jnp.float32),
            pltpu.SemaphoreType.DMA((2,)),
        ],
    )
    def gather_kernel(tabt_hbm, idx_hbm, out_hbm, tabt_v, idx_v, stg_v, ssem):
        cid = lax.axis_index("c")
        sid = lax.axis_index("s")
        wid = sid * nc + cid

        pltpu.sync_copy(tabt_hbm, tabt_v)
        pltpu.sync_copy(idx_hbm.at[pl.ds(wid * nb * H, nb * H)], idx_v)

        iota = lax.iota(jnp.int32, L)

        def store_dst(h):
            return out_hbm.at[h, :, pl.ds(wid * nbt, nbt)]

        def start_store(p, h):
            pltpu.async_copy(stg_v.at[p], store_dst(h), ssem.at[p])

        def wait_store(p, h):
            pltpu.make_async_copy(stg_v.at[p], store_dst(h),
                                  ssem.at[p]).wait()

        def produce(p, h):
            for btl in range(nbt):
                for g in range(128 // L):
                    tvec = (iota + (btl * 128 + g * L)) * H
                    idxvals = plsc.load_gather(idx_v, [tvec + h])
                    for d in range(D):
                        vals = plsc.load_gather(
                            tabt_v, [jnp.full((L,), d, jnp.int32), idxvals])
                        stg_v[p, d // di, btl, d % di, pl.ds(g * L, L)] = vals

        def block(j, carry):
            for p in range(2):
                h = 2 * j + p

                @pl.when(j > 0)
                def _():
                    wait_store(p, h - 2)

                produce(p, h)
                start_store(p, h)
            return carry

        lax.fori_loop(0, H // 2, block, 0)
        wait_store(0, H - 2)
        wait_store(1, H - 1)

    return gather_kernel


def kernel(inputs, table):
    batch, hist = inputs.shape
    rows, dim = table.shape
    flat_idx = inputs.reshape(batch * hist)
    table_t = jnp.transpose(table)
    out5 = _make_gather(batch, hist, rows, dim)(table_t, flat_idx)
    out = jnp.transpose(out5, (2, 4, 0, 1, 3))
    return out.reshape(batch, hist, dim)


# speedup vs baseline: 9.2105x; 1.0032x over previous
import functools

import jax
import jax.numpy as jnp
from jax import lax
from jax.experimental import pallas as pl
from jax.experimental.pallas import tpu as pltpu
from jax.experimental.pallas import tpu_sc as plsc

L = 16


@functools.cache
def _make_gather(B: int, H: int, V: int, D: int):
    info = plsc.get_sparse_core_info()
    nc, ns = info.num_cores, info.num_subcores
    nw = nc * ns
    nb = B // nw
    nbt = nb // 128
    dt, di = D // 8, 8
    assert nb * nw == B and nbt * 128 == nb
    mesh = plsc.VectorSubcoreMesh(core_axis_name="c", subcore_axis_name="s")

    @functools.partial(
        pl.kernel,
        mesh=mesh,
        out_type=jax.ShapeDtypeStruct((H, dt, B // 128, di, 128), jnp.float32),
        compiler_params=pltpu.CompilerParams(
            use_tc_tiling_on_sc=False, needs_layout_passes=False,
            disable_bounds_checks=True),
        scratch_types=[
            pltpu.VMEM((D, V), jnp.float32),
            pltpu.VMEM((nb * H,), jnp.int32),
            pltpu.VMEM((2, dt, nbt, di, 128), jnp.float32),
            pltpu.SemaphoreType.DMA((2,)),
        ],
    )
    def gather_kernel(tabt_hbm, idx_hbm, out_hbm, tabt_v, idx_v, stg_v, ssem):
        cid = lax.axis_index("c")
        sid = lax.axis_index("s")
        wid = sid * nc + cid

        pltpu.sync_copy(tabt_hbm, tabt_v)
        pltpu.sync_copy(idx_hbm.at[pl.ds(wid * nb * H, nb * H)], idx_v)

        iota = lax.iota(jnp.int32, L)

        def store_dst(h):
            return out_hbm.at[h, :, pl.ds(wid * nbt, nbt)]

        def start_store(p, h):
            pltpu.async_copy(stg_v.at[p], store_dst(h), ssem.at[p])

        def wait_store(p, h):
            pltpu.make_async_copy(stg_v.at[p], store_dst(h),
                                  ssem.at[p]).wait()

        def produce(p, h):
            for btl in range(nbt):
                for g in range(128 // L):
                    tvec = (iota + (btl * 128 + g * L)) * H
                    idxvals = plsc.load_gather(idx_v, [tvec + h])
                    for d in range(D):
                        vals = plsc.load_gather(
                            tabt_v, [jnp.full((L,), d, jnp.int32), idxvals])
                        stg_v[p, d // di, btl, d % di, pl.ds(g * L, L)] = vals

        def block(j, carry):
            for p in range(2):
                h = 2 * j + p

                @pl.when(j > 0)
                def _():
                    wait_store(p, h - 2)

                produce(p, h)
                start_store(p, h)
            return carry

        lax.fori_loop(0, H // 2, block, 0)
        wait_store(0, H - 2)
        wait_store(1, H - 1)

    return gather_kernel


def kernel(inputs, table):
    batch, hist = inputs.shape
    rows, dim = table.shape
    flat_idx = inputs.reshape(batch * hist)
    table_t = jnp.transpose(table)
    out5 = _make_gather(batch, hist, rows, dim)(table_t, flat_idx)
    out = jnp.transpose(out5, (2, 4, 0, 1, 3))
    return out.reshape(batch, hist, dim)


# speedup vs baseline: 9.3829x; 1.0187x over previous
import functools

import jax
import jax.numpy as jnp
from jax import lax
from jax.experimental import pallas as pl
from jax.experimental.pallas import tpu as pltpu
from jax.experimental.pallas import tpu_sc as plsc

L = 16


@functools.cache
def _make_gather(B: int, H: int, V: int, D: int):
    info = plsc.get_sparse_core_info()
    nc, ns = info.num_cores, info.num_subcores
    nw = nc * ns
    nb = B // nw
    nbt = nb // 128
    dt, di = D // 8, 8
    assert nb * nw == B and nbt * 128 == nb
    mesh = plsc.VectorSubcoreMesh(core_axis_name="c", subcore_axis_name="s")

    @functools.partial(
        pl.kernel,
        mesh=mesh,
        out_type=jax.ShapeDtypeStruct((H, dt, B // 128, di, 128), jnp.float32),
        compiler_params=pltpu.CompilerParams(
            use_tc_tiling_on_sc=False, needs_layout_passes=False,
            disable_bounds_checks=True),
        scratch_types=[
            pltpu.VMEM((V * D * 8,), jnp.float32),
            pltpu.VMEM((nb * H,), jnp.int32),
            pltpu.VMEM((2, dt, nbt, di, 128), jnp.float32),
            pltpu.SemaphoreType.DMA((2,)),
        ],
    )
    def gather_kernel(tab8_hbm, idx_hbm, out_hbm, tab8_v, idx_v, stg_v, ssem):
        cid = lax.axis_index("c")
        sid = lax.axis_index("s")
        wid = sid * nc + cid

        pltpu.sync_copy(tab8_hbm, tab8_v)
        pltpu.sync_copy(idx_hbm.at[pl.ds(wid * nb * H, nb * H)], idx_v)

        iota = lax.iota(jnp.int32, L)
        iota8 = lax.rem(iota, jnp.full((L,), 8, jnp.int32))

        def store_dst(h):
            return out_hbm.at[h, :, pl.ds(wid * nbt, nbt)]

        def start_store(p, h):
            pltpu.async_copy(stg_v.at[p], store_dst(h), ssem.at[p])

        def wait_store(p, h):
            pltpu.make_async_copy(stg_v.at[p], store_dst(h),
                                  ssem.at[p]).wait()

        def produce(p, h):
            for btl in range(nbt):
                for g in range(128 // L):
                    tvec = (iota + (btl * 128 + g * L)) * H
                    idxvals = plsc.load_gather(idx_v, [tvec + h])
                    base = idxvals * (D * 8) + iota8
                    for d in range(D):
                        vals = plsc.load_gather(tab8_v, [base + d * 8])
                        stg_v[p, d // di, btl, d % di, pl.ds(g * L, L)] = vals

        def block(j, carry):
            for p in range(2):
                h = 2 * j + p

                @pl.when(j > 0)
                def _():
                    wait_store(p, h - 2)

                produce(p, h)
                start_store(p, h)
            return carry

        lax.fori_loop(0, H // 2, block, 0)
        wait_store(0, H - 2)
        wait_store(1, H - 1)

    return gather_kernel


def kernel(inputs, table):
    batch, hist = inputs.shape
    rows, dim = table.shape
    flat_idx = inputs.reshape(batch * hist)
    table_8 = jnp.tile(table.reshape(rows * dim, 1), (1, 8)).reshape(-1)
    out5 = _make_gather(batch, hist, rows, dim)(table_8, flat_idx)
    out = jnp.transpose(out5, (2, 4, 0, 1, 3))
    return out.reshape(batch, hist, dim)


# speedup vs baseline: 21.2102x; 2.2605x over previous
import functools

import jax
import jax.numpy as jnp
from jax import lax
from jax.experimental import pallas as pl
from jax.experimental.pallas import tpu as pltpu
from jax.experimental.pallas import tpu_sc as plsc

L = 16


@functools.cache
def _make_gather(B: int, H: int, V: int, D: int):
    info = plsc.get_sparse_core_info()
    nc, ns = info.num_cores, info.num_subcores
    nw = nc * ns
    nb = B // nw
    nbt = nb // 128
    dt, di = D // 8, 8
    assert nb * nw == B and nbt * 128 == nb
    mesh = plsc.VectorSubcoreMesh(core_axis_name="c", subcore_axis_name="s")

    @functools.partial(
        pl.kernel,
        mesh=mesh,
        out_type=jax.ShapeDtypeStruct((H, dt, B // 128, di, 128), jnp.float32),
        compiler_params=pltpu.CompilerParams(
            use_tc_tiling_on_sc=False, needs_layout_passes=False,
            disable_bounds_checks=True),
        scratch_types=[
            pltpu.VMEM((V * D * 8,), jnp.float32),
            pltpu.VMEM((nb * H,), jnp.int32),
            pltpu.VMEM((2, dt, nbt, di, 128), jnp.float32),
            pltpu.SemaphoreType.DMA((2,)),
        ],
    )
    def gather_kernel(tab8_hbm, idx_hbm, out_hbm, tab8_v, idx_v, stg_v, ssem):
        cid = lax.axis_index("c")
        sid = lax.axis_index("s")
        wid = sid * nc + cid

        pltpu.sync_copy(tab8_hbm, tab8_v)
        pltpu.sync_copy(idx_hbm.at[pl.ds(wid * nb * H, nb * H)], idx_v)

        iota = lax.iota(jnp.int32, L)
        iota8 = lax.rem(iota, jnp.full((L,), 8, jnp.int32))
        iota_h = iota * H

        def store_dst(h):
            return out_hbm.at[h, :, pl.ds(wid * nbt, nbt)]

        def start_store(p, h):
            pltpu.async_copy(stg_v.at[p], store_dst(h), ssem.at[p])

        def wait_store(p, h):
            pltpu.make_async_copy(stg_v.at[p], store_dst(h),
                                  ssem.at[p]).wait()

        def produce(p, h):
            @plsc.parallel_loop(0, nbt * (128 // L), 1, unroll=2)
            def _(c):
                btl = lax.shift_right_logical(c, 3)
                g = lax.bitwise_and(c, 7)
                t0 = btl * 128 + g * L
                idxvals = plsc.load_gather(idx_v, [iota_h + (t0 * H + h)])
                base = idxvals * (D * 8) + iota8
                for d in range(D):
                    vals = plsc.load_gather(tab8_v, [base + d * 8])
                    stg_v[p, d // di, btl, d % di, pl.ds(g * L, L)] = vals

        def block(j, carry):
            for p in range(2):
                h = 2 * j + p

                @pl.when(j > 0)
                def _():
                    wait_store(p, h - 2)

                produce(p, h)
                start_store(p, h)
            return carry

        lax.fori_loop(0, H // 2, block, 0)
        wait_store(0, H - 2)
        wait_store(1, H - 1)

    return gather_kernel


def kernel(inputs, table):
    batch, hist = inputs.shape
    rows, dim = table.shape
    flat_idx = inputs.reshape(batch * hist)
    table_8 = jnp.tile(table.reshape(rows * dim, 1), (1, 8)).reshape(-1)
    out5 = _make_gather(batch, hist, rows, dim)(table_8, flat_idx)
    out = jnp.transpose(out5, (2, 4, 0, 1, 3))
    return out.reshape(batch, hist, dim)


# speedup vs baseline: 22.1381x; 1.0437x over previous
import functools

import jax
import jax.numpy as jnp
from jax import lax
from jax.experimental import pallas as pl
from jax.experimental.pallas import tpu as pltpu
from jax.experimental.pallas import tpu_sc as plsc

L = 16


@functools.cache
def _make_gather(B: int, H: int, V: int, D: int):
    info = plsc.get_sparse_core_info()
    nc, ns = info.num_cores, info.num_subcores
    nw = nc * ns
    nb = B // nw
    nbt = nb // 128
    dt, di = D // 8, 8
    assert nb * nw == B and nbt * 128 == nb
    mesh = plsc.VectorSubcoreMesh(core_axis_name="c", subcore_axis_name="s")

    @functools.partial(
        pl.kernel,
        mesh=mesh,
        out_type=jax.ShapeDtypeStruct((H, dt, B // 128, di, 128), jnp.float32),
        compiler_params=pltpu.CompilerParams(
            use_tc_tiling_on_sc=False, needs_layout_passes=False,
            disable_bounds_checks=True),
        scratch_types=[
            pltpu.VMEM((V * D * 8,), jnp.float32),
            pltpu.VMEM((nb * H,), jnp.int32),
            pltpu.VMEM((2, dt, nbt, di, 128), jnp.float32),
            pltpu.SemaphoreType.DMA((2,)),
        ],
    )
    def gather_kernel(tab8_hbm, idx_hbm, out_hbm, tab8_v, idx_v, stg_v, ssem):
        cid = lax.axis_index("c")
        sid = lax.axis_index("s")
        wid = sid * nc + cid

        pltpu.sync_copy(tab8_hbm, tab8_v)
        pltpu.sync_copy(idx_hbm.at[pl.ds(wid * nb * H, nb * H)], idx_v)

        iota = lax.iota(jnp.int32, L)
        iota8 = lax.rem(iota, jnp.full((L,), 8, jnp.int32))
        iota_h = iota * H

        def store_dst(h):
            return out_hbm.at[h, :, pl.ds(wid * nbt, nbt)]

        def start_store(p, h):
            pltpu.async_copy(stg_v.at[p], store_dst(h), ssem.at[p])

        def wait_store(p, h):
            pltpu.make_async_copy(stg_v.at[p], store_dst(h),
                                  ssem.at[p]).wait()

        def produce(p, h):
            @plsc.parallel_loop(0, nbt * (128 // L), 1, unroll=4)
            def _(c):
                btl = lax.shift_right_logical(c, 3)
                g = lax.bitwise_and(c, 7)
                t0 = btl * 128 + g * L
                idxvals = plsc.load_gather(idx_v, [iota_h + (t0 * H + h)])
                base = idxvals * (D * 8) + iota8
                for d in range(D):
                    vals = plsc.load_gather(tab8_v, [base + d * 8])
                    stg_v[p, d // di, btl, d % di, pl.ds(g * L, L)] = vals

        def block(j, carry):
            for p in range(2):
                h = 2 * j + p

                @pl.when(j > 0)
                def _():
                    wait_store(p, h - 2)

                produce(p, h)
                start_store(p, h)
            return carry

        lax.fori_loop(0, H // 2, block, 0)
        wait_store(0, H - 2)
        wait_store(1, H - 1)

    return gather_kernel


def kernel(inputs, table):
    batch, hist = inputs.shape
    rows, dim = table.shape
    flat_idx = inputs.reshape(batch * hist)
    table_8 = jnp.tile(table.reshape(rows * dim, 1), (1, 8)).reshape(-1)
    out5 = _make_gather(batch, hist, rows, dim)(table_8, flat_idx)
    out = jnp.transpose(out5, (2, 4, 0, 1, 3))
    return out.reshape(batch, hist, dim)
